# baseline (speedup 1.0000x reference)
import functools

import jax
import jax.numpy as jnp
from jax import lax
from jax.experimental import pallas as pl
from jax.experimental.pallas import tpu as pltpu
from jax.experimental.pallas import tpu_sc as plsc

NUM_CORES = 2
NUM_SUBCORES = 16
NUM_WORKERS = NUM_CORES * NUM_SUBCORES
GROUP = 128
K = 2
NBUF = 4
LANES = 16


@functools.partial(jax.jit, static_argnames=("hist", "d"))
def _sc_gather_t(idx, table, *, hist, d):
    total = idx.shape[0]
    batch = total // hist
    nblk = batch // GROUP
    n_units = hist * nblk
    units_per_w = n_units // NUM_WORKERS
    n_chunks = units_per_w // K
    n_rounds = n_chunks // NBUF
    assert n_chunks == n_rounds * NBUF and n_rounds >= 3
    assert d == 32

    mesh = plsc.VectorSubcoreMesh(
        core_axis_name="c", subcore_axis_name="s",
        num_cores=NUM_CORES, num_subcores=NUM_SUBCORES,
    )

    @functools.partial(
        pl.kernel,
        out_type=jax.ShapeDtypeStruct((hist, d // 8, nblk, 8, GROUP),
                                      jnp.float32),
        mesh=mesh,
        scratch_types=[
            [pltpu.VMEM((K * GROUP,), jnp.int32) for _ in range(NBUF)],
            [pltpu.VMEM((K, GROUP, d), jnp.float32) for _ in range(NBUF)],
            [pltpu.VMEM((K, d, GROUP + 5), jnp.float32) for _ in range(NBUF)],
            [pltpu.SemaphoreType.DMA for _ in range(NBUF)],
            [pltpu.SemaphoreType.DMA for _ in range(NBUF)],
            [pltpu.SemaphoreType.DMA for _ in range(NBUF)],
        ],
        compiler_params=pltpu.CompilerParams(
            use_tc_tiling_on_sc=False, needs_layout_passes=False),
    )
    def gather_kernel(idx_hbm, table_hbm, out_hbm, ivs, rvs, tvs,
                      isems, gsems, osems):
        wid = lax.axis_index("s") * NUM_CORES + lax.axis_index("c")
        ubase = wid * units_per_w
        lane_iota = lax.iota(jnp.int32, LANES)

        def fire_idx(ch, q):
            pltpu.make_async_copy(
                idx_hbm.at[pl.ds((ubase + ch * K) * GROUP, K * GROUP)],
                ivs[q], isems[q]).start()

        def wait_idx(q):
            pltpu.make_async_copy(
                idx_hbm.at[pl.ds(ubase * GROUP, K * GROUP)],
                ivs[q], isems[q]).wait()

        def fire_gathers(b):
            for j in range(K):
                pltpu.make_async_copy(
                    table_hbm.at[ivs[b].at[pl.ds(j * GROUP, GROUP)]],
                    rvs[b].at[j], gsems[b]).start()

        def wait_gathers(q):
            pltpu.make_async_copy(
                out_hbm.at[0].at[pl.ds(0, K)], rvs[q], gsems[q]).wait()

        zero16 = lane_iota * 0

        def transpose_chunk(q):
            for j in range(K):
                dst = tvs[q].at[j]

                @plsc.parallel_loop(0, GROUP, unroll=8)
                def _(l):
                    l_vec = zero16 + l
                    for g in range(d // LANES):
                        vec = rvs[q][j, l, pl.ds(g * LANES, LANES)]
                        plsc.store_scatter(
                            dst, [g * LANES + lane_iota, l_vec], vec)

        def fire_writes(ch, q):
            u0 = ubase + ch * K
            for j in range(K):
                u = u0 + j
                h = u // nblk
                c = lax.rem(u, nblk)
                for r in range(d // 8):
                    pltpu.make_async_copy(
                        tvs[q].at[j, pl.ds(r * 8, 8), pl.ds(0, GROUP)],
                        out_hbm.at[h, r, c], osems[q]).start()

        def wait_writes(q):
            for _ in range(K * (d // 8)):
                pltpu.make_async_copy(
                    tvs[q].at[0, pl.ds(0, 8), pl.ds(0, GROUP)],
                    out_hbm.at[0, 0, 0], osems[q]).wait()

        fire_idx(0, 0)
        for b in range(NBUF):
            if b >= 2:
                wait_gathers(b - 2)
                transpose_chunk(b - 2)
                fire_writes(b - 2, b - 2)
            wait_idx(b)
            fire_gathers(b)
            fire_idx(b + 1, (b + 1) % NBUF)

        def round_body(r, carry):
            v0 = r * NBUF
            for b in range(NBUF):
                v = v0 + b
                q = (b + 2) % NBUF
                wait_gathers(q)
                transpose_chunk(q)
                fire_writes(v - 2, q)
                wait_writes(b)
                wait_idx(b)
                fire_gathers(b)
                fire_idx(v + 1, (b + 1) % NBUF)
            return carry

        lax.fori_loop(1, n_rounds - 1, round_body, 0)

        v0 = (n_rounds - 1) * NBUF
        for b in range(NBUF):
            v = v0 + b
            q = (b + 2) % NBUF
            wait_gathers(q)
            transpose_chunk(q)
            fire_writes(v - 2, q)
            wait_writes(b)
            wait_idx(b)
            fire_gathers(b)
            if b + 1 < NBUF:
                fire_idx(v + 1, b + 1)

        n = n_chunks
        wait_gathers(2)
        transpose_chunk(2)
        fire_writes(n - 2, 2)
        wait_gathers(3)
        transpose_chunk(3)
        fire_writes(n - 1, 3)
        for q in range(NBUF):
            wait_writes(q)

    return gather_kernel(idx, table)


@functools.partial(jax.jit, static_argnames=("d",))
def _sc_table_transpose(tab_t, *, d):
    v = tab_t.shape[1]
    blk = 2 * GROUP
    n_full = v // blk
    base_per_w = n_full // NUM_WORKERS
    n_extra = n_full - base_per_w * NUM_WORKERS
    n_pairs = (base_per_w + 2) // 2

    mesh = plsc.VectorSubcoreMesh(
        core_axis_name="c", subcore_axis_name="s",
        num_cores=NUM_CORES, num_subcores=NUM_SUBCORES,
    )

    @functools.partial(
        pl.kernel,
        out_type=jax.ShapeDtypeStruct((v * d,), jnp.float32),
        mesh=mesh,
        scratch_types=[
            [pltpu.VMEM((d, blk + 5), jnp.float32) for _ in range(2)],
            [pltpu.VMEM((blk * d,), jnp.float32) for _ in range(2)],
            [pltpu.SemaphoreType.DMA for _ in range(2)],
            [pltpu.SemaphoreType.DMA for _ in range(2)],
        ],
        compiler_params=pltpu.CompilerParams(
            use_tc_tiling_on_sc=True, needs_layout_passes=False),
    )
    def transpose_kernel(tab_hbm, out_hbm, spads, flats, isems, osems):
        wid = lax.axis_index("s") * NUM_CORES + lax.axis_index("c")
        n_w = base_per_w + jnp.where(wid < n_extra, 1, 0)
        lane_iota = lax.iota(jnp.int32, LANES)
        zero16 = lane_iota * 0

        def col_of(k):
            return wid + NUM_WORKERS * k

        def fire_in(k, b):
            pltpu.make_async_copy(
                tab_hbm.at[:, pl.ds(col_of(k) * blk, blk)],
                spads[b].at[:, pl.ds(0, blk)], isems[b]).start()

        def wait_in(b):
            pltpu.make_async_copy(
                tab_hbm.at[:, pl.ds(0, blk)],
                spads[b].at[:, pl.ds(0, blk)], isems[b]).wait()

        def transpose_block(b):
            spad = spads[b]
            flat = flats[b]

            @plsc.parallel_loop(0, blk, unroll=8)
            def _(l):
                l_vec = zero16 + l
                for g in range(d // LANES):
                    vec = plsc.load_gather(
                        spad, [g * LANES + lane_iota, l_vec])
                    flat[pl.ds(l * d + g * LANES, LANES)] = vec

        def fire_out(k, b):
            pltpu.make_async_copy(
                flats[b], out_hbm.at[pl.ds(col_of(k) * blk * d, blk * d)],
                osems[b]).start()

        def wait_out(b):
            pltpu.make_async_copy(
                flats[b], out_hbm.at[pl.ds(0, blk * d)], osems[b]).wait()

        fire_in(0, 0)
        fire_in(1, 1)

        def pair_body(r, carry):
            for b in range(2):
                k = 2 * r + b

                @pl.when(k < n_w)
                def _():
                    wait_in(b)

                    @pl.when(k >= 2)
                    def _():
                        wait_out(b)

                    transpose_block(b)
                    fire_out(k, b)

                    @pl.when(k + 2 < n_w)
                    def _():
                        fire_in(k + 2, b)
            return carry

        lax.fori_loop(0, n_pairs, pair_body, 0)
        wait_out(0)
        wait_out(1)

    return transpose_kernel(tab_t)


def kernel(x, table):
    batch, hist = x.shape
    vocab, d = table.shape
    idx = x.T.reshape(batch * hist).astype(jnp.int32)
    t_lin = _sc_table_transpose(table.T, d=d)
    n_full = vocab // (2 * GROUP) * (2 * GROUP)
    if n_full < vocab:
        tail_flat = lax.slice(table, (n_full, 0), (vocab, d)).reshape(-1)
        t_lin = lax.dynamic_update_slice(t_lin, tail_flat, (n_full * d,))
    t_rm = t_lin.reshape(vocab, d)
    t5 = _sc_gather_t(idx, t_rm, hist=hist, d=d)
    out = t5.transpose(2, 4, 0, 1, 3).reshape(batch, hist, d)
    return out


# speedup vs baseline: 1.0053x; 1.0053x over previous
import functools

import jax
import jax.numpy as jnp
from jax import lax
from jax.experimental import pallas as pl
from jax.experimental.pallas import tpu as pltpu
from jax.experimental.pallas import tpu_sc as plsc

NUM_CORES = 2
NUM_SUBCORES = 16
NUM_WORKERS = NUM_CORES * NUM_SUBCORES
GROUP = 128
K = 2
NBUF = 4
LANES = 16


@functools.partial(jax.jit, static_argnames=("hist", "d"))
def _sc_gather_t(idx, table, *, hist, d):
    total = idx.shape[0]
    batch = total // hist
    nblk = batch // GROUP
    n_units = hist * nblk
    units_per_w = n_units // NUM_WORKERS
    n_chunks = units_per_w // K
    n_rounds = n_chunks // NBUF
    assert n_chunks == n_rounds * NBUF and n_rounds >= 3
    assert d == 32

    mesh = plsc.VectorSubcoreMesh(
        core_axis_name="c", subcore_axis_name="s",
        num_cores=NUM_CORES, num_subcores=NUM_SUBCORES,
    )

    @functools.partial(
        pl.kernel,
        out_type=jax.ShapeDtypeStruct((hist, d // 8, nblk, 8, GROUP),
                                      jnp.float32),
        mesh=mesh,
        scratch_types=[
            [pltpu.VMEM((K * GROUP,), jnp.int32) for _ in range(NBUF)],
            [pltpu.VMEM((K, GROUP, d), jnp.float32) for _ in range(NBUF)],
            [pltpu.VMEM((K, d, GROUP + 5), jnp.float32) for _ in range(NBUF)],
            [pltpu.SemaphoreType.DMA for _ in range(NBUF)],
            [pltpu.SemaphoreType.DMA for _ in range(NBUF)],
            [pltpu.SemaphoreType.DMA for _ in range(NBUF)],
        ],
        compiler_params=pltpu.CompilerParams(
            use_tc_tiling_on_sc=False, needs_layout_passes=False),
    )
    def gather_kernel(idx_hbm, table_hbm, out_hbm, ivs, rvs, tvs,
                      isems, gsems, osems):
        wid = lax.axis_index("s") * NUM_CORES + lax.axis_index("c")
        ubase = wid * units_per_w
        lane_iota = lax.iota(jnp.int32, LANES)

        def fire_idx(ch, q):
            pltpu.make_async_copy(
                idx_hbm.at[pl.ds((ubase + ch * K) * GROUP, K * GROUP)],
                ivs[q], isems[q]).start()

        def wait_idx(q):
            pltpu.make_async_copy(
                idx_hbm.at[pl.ds(ubase * GROUP, K * GROUP)],
                ivs[q], isems[q]).wait()

        def fire_gathers(b):
            for j in range(K):
                pltpu.make_async_copy(
                    table_hbm.at[ivs[b].at[pl.ds(j * GROUP, GROUP)]],
                    rvs[b].at[j], gsems[b]).start()

        def wait_gathers(q):
            pltpu.make_async_copy(
                out_hbm.at[0].at[pl.ds(0, K)], rvs[q], gsems[q]).wait()

        zero16 = lane_iota * 0

        def transpose_chunk(q):
            for j in range(K):
                dst = tvs[q].at[j]

                @plsc.parallel_loop(0, GROUP, unroll=8)
                def _(l):
                    l_vec = zero16 + l
                    for g in range(d // LANES):
                        vec = rvs[q][j, l, pl.ds(g * LANES, LANES)]
                        plsc.store_scatter(
                            dst, [g * LANES + lane_iota, l_vec], vec)

        def fire_writes(ch, q):
            u0 = ubase + ch * K
            for j in range(K):
                u = u0 + j
                h = u // nblk
                c = lax.rem(u, nblk)
                for r in range(d // 8):
                    pltpu.make_async_copy(
                        tvs[q].at[j, pl.ds(r * 8, 8), pl.ds(0, GROUP)],
                        out_hbm.at[h, r, c], osems[q]).start()

        def wait_writes(q):
            for _ in range(K * (d // 8)):
                pltpu.make_async_copy(
                    tvs[q].at[0, pl.ds(0, 8), pl.ds(0, GROUP)],
                    out_hbm.at[0, 0, 0], osems[q]).wait()

        fire_idx(0, 0)
        for b in range(NBUF):
            if b >= 2:
                wait_gathers(b - 2)
                transpose_chunk(b - 2)
                fire_writes(b - 2, b - 2)
            wait_idx(b)
            fire_gathers(b)
            fire_idx(b + 1, (b + 1) % NBUF)

        def round_body(r, carry):
            v0 = r * NBUF
            for b in range(NBUF):
                v = v0 + b
                q = (b + 2) % NBUF
                wait_gathers(q)
                transpose_chunk(q)
                fire_writes(v - 2, q)
                wait_writes(b)
                wait_idx(b)
                fire_gathers(b)
                fire_idx(v + 1, (b + 1) % NBUF)
            return carry

        lax.fori_loop(1, n_rounds - 1, round_body, 0)

        v0 = (n_rounds - 1) * NBUF
        for b in range(NBUF):
            v = v0 + b
            q = (b + 2) % NBUF
            wait_gathers(q)
            transpose_chunk(q)
            fire_writes(v - 2, q)
            wait_writes(b)
            wait_idx(b)
            fire_gathers(b)
            if b + 1 < NBUF:
                fire_idx(v + 1, b + 1)

        n = n_chunks
        wait_gathers(2)
        transpose_chunk(2)
        fire_writes(n - 2, 2)
        wait_gathers(3)
        transpose_chunk(3)
        fire_writes(n - 1, 3)
        for q in range(NBUF):
            wait_writes(q)

    return gather_kernel(idx, table)


@functools.partial(jax.jit, static_argnames=("d",))
def _sc_table_transpose(tab_t, *, d):
    v = tab_t.shape[1]
    blk = GROUP
    n_full = v // blk
    base_per_w = n_full // NUM_WORKERS
    n_extra = n_full - base_per_w * NUM_WORKERS
    n_pairs = (base_per_w + 2) // 2

    mesh = plsc.VectorSubcoreMesh(
        core_axis_name="c", subcore_axis_name="s",
        num_cores=NUM_CORES, num_subcores=NUM_SUBCORES,
    )

    @functools.partial(
        pl.kernel,
        out_type=jax.ShapeDtypeStruct((v * d,), jnp.float32),
        mesh=mesh,
        scratch_types=[
            [pltpu.VMEM((d, blk + 5), jnp.float32) for _ in range(2)],
            [pltpu.VMEM((blk * d,), jnp.float32) for _ in range(2)],
            [pltpu.SemaphoreType.DMA for _ in range(2)],
            [pltpu.SemaphoreType.DMA for _ in range(2)],
        ],
        compiler_params=pltpu.CompilerParams(
            use_tc_tiling_on_sc=True, needs_layout_passes=False),
    )
    def transpose_kernel(tab_hbm, out_hbm, spads, flats, isems, osems):
        wid = lax.axis_index("s") * NUM_CORES + lax.axis_index("c")
        n_w = base_per_w + jnp.where(wid < n_extra, 1, 0)
        lane_iota = lax.iota(jnp.int32, LANES)
        zero16 = lane_iota * 0

        def col_of(k):
            return wid + NUM_WORKERS * k

        def fire_in(k, b):
            pltpu.make_async_copy(
                tab_hbm.at[:, pl.ds(col_of(k) * blk, blk)],
                spads[b].at[:, pl.ds(0, blk)], isems[b]).start()

        def wait_in(b):
            pltpu.make_async_copy(
                tab_hbm.at[:, pl.ds(0, blk)],
                spads[b].at[:, pl.ds(0, blk)], isems[b]).wait()

        def transpose_block(b):
            spad = spads[b]
            flat = flats[b]

            @plsc.parallel_loop(0, blk, unroll=8)
            def _(l):
                l_vec = zero16 + l
                for g in range(d // LANES):
                    vec = plsc.load_gather(
                        spad, [g * LANES + lane_iota, l_vec])
                    flat[pl.ds(l * d + g * LANES, LANES)] = vec

        def fire_out(k, b):
            pltpu.make_async_copy(
                flats[b], out_hbm.at[pl.ds(col_of(k) * blk * d, blk * d)],
                osems[b]).start()

        def wait_out(b):
            pltpu.make_async_copy(
                flats[b], out_hbm.at[pl.ds(0, blk * d)], osems[b]).wait()

        fire_in(0, 0)
        fire_in(1, 1)

        def pair_body(r, carry):
            for b in range(2):
                k = 2 * r + b

                @pl.when(k < n_w)
                def _():
                    wait_in(b)

                    @pl.when(k >= 2)
                    def _():
                        wait_out(b)

                    transpose_block(b)
                    fire_out(k, b)

                    @pl.when(k + 2 < n_w)
                    def _():
                        fire_in(k + 2, b)
            return carry

        lax.fori_loop(0, n_pairs, pair_body, 0)
        wait_out(0)
        wait_out(1)

    return transpose_kernel(tab_t)


def kernel(x, table):
    batch, hist = x.shape
    vocab, d = table.shape
    idx = x.T.reshape(batch * hist).astype(jnp.int32)
    t_lin = _sc_table_transpose(table.T, d=d)
    n_full = vocab // GROUP * GROUP
    if n_full < vocab:
        tail_flat = lax.slice(table, (n_full, 0), (vocab, d)).reshape(-1)
        t_lin = lax.dynamic_update_slice(t_lin, tail_flat, (n_full * d,))
    t_rm = t_lin.reshape(vocab, d)
    t5 = _sc_gather_t(idx, t_rm, hist=hist, d=d)
    out = t5.transpose(2, 4, 0, 1, 3).reshape(batch, hist, d)
    return out
